# Initial kernel scaffold; baseline (speedup 1.0000x reference)
import jax
import jax.numpy as jnp
from jax.experimental import pallas as pl


def kernel(pts, voxel_features):
    raise NotImplementedError("write your pallas kernel here")


# speedup vs baseline: 31.0860x; 31.0860x over previous
import functools

import jax
import jax.numpy as jnp
from jax import lax
from jax.experimental import pallas as pl
from jax.experimental.pallas import tpu as pltpu
from jax.experimental.pallas import tpu_sc as plsc

RES_INV = 1024.0
BUCKETS = 4194304
HASH_MASK = BUCKETS - 1
P2 = 2654435761
P3 = 805459861
N_PTS = 2097152
D = 2

NUM_CORES = 2
NUM_SUBCORES = 16
LANES = 16
NUM_WORKERS = NUM_CORES * NUM_SUBCORES
PTS_PER_WORKER = N_PTS // NUM_WORKERS
CHUNK = 1024
N_CHUNKS = PTS_PER_WORKER // CHUNK


def _make_sc_kernel():
    mesh = plsc.VectorSubcoreMesh(core_axis_name="c", subcore_axis_name="s")

    scratch = (
        [pltpu.VMEM((CHUNK,), jnp.float32) for _ in range(3)]
        + [pltpu.VMEM((CHUNK,), jnp.int32) for _ in range(8)]
        + [pltpu.VMEM((CHUNK,), jnp.float32) for _ in range(8)]
        + [pltpu.VMEM((CHUNK, D), jnp.float32) for _ in range(8)]
        + [
            pltpu.VMEM((CHUNK * D,), jnp.float32),
            pltpu.SemaphoreType.DMA,
            pltpu.SemaphoreType.DMA,
        ]
    )

    @functools.partial(
        pl.kernel,
        out_type=jax.ShapeDtypeStruct((N_PTS * D,), jnp.float32),
        mesh=mesh,
        scratch_types=scratch,
        compiler_params=pltpu.CompilerParams(
            needs_layout_passes=False,
            use_tc_tiling_on_sc=False,
        ),
    )
    def sc_kernel(xs_hbm, ys_hbm, zs_hbm, table_hbm, out_hbm, *refs):
        pts_hbm = (xs_hbm, ys_hbm, zs_hbm)
        xyz_v = refs[0:3]
        idx_v = refs[3:11]
        w_v = refs[11:19]
        rows_v = refs[19:27]
        acc_v, sem_in, sem_g = refs[27:30]

        wid = lax.axis_index("s") * NUM_CORES + lax.axis_index("c")
        lane = lax.iota(jnp.int32, LANES)
        pair = lane >> 1
        feat = lane & 1

        def chunk_body(c, _):
            base = wid * PTS_PER_WORKER + c * CHUNK
            for a in range(3):
                pltpu.async_copy(
                    pts_hbm[a].at[pl.ds(base, CHUNK)], xyz_v[a], sem_in
                ).wait()

            def hash_body(j, _):
                o = j * LANES
                qx = xyz_v[0][pl.ds(o, LANES)] * RES_INV
                qy = xyz_v[1][pl.ds(o, LANES)] * RES_INV
                qz = xyz_v[2][pl.ds(o, LANES)] * RES_INV
                bx = qx.astype(jnp.int32)
                by = qy.astype(jnp.int32)
                bz = qz.astype(jnp.int32)
                fx = qx - bx.astype(jnp.float32)
                fy = qy - by.astype(jnp.float32)
                fz = qz - bz.astype(jnp.float32)
                hx = (bx.astype(jnp.uint32), bx.astype(jnp.uint32) + jnp.uint32(1))
                hy0 = by.astype(jnp.uint32) * jnp.uint32(P2)
                hy = (hy0, hy0 + jnp.uint32(P2))
                hz0 = bz.astype(jnp.uint32) * jnp.uint32(P3)
                hz = (hz0, hz0 + jnp.uint32(P3))
                wx = (1.0 - fx, fx)
                wy = (1.0 - fy, fy)
                wz = (1.0 - fz, fz)
                for k in range(8):
                    kx, ky, kz = k & 1, (k >> 1) & 1, k >> 2
                    h = (hx[kx] ^ hy[ky] ^ hz[kz]) & jnp.uint32(HASH_MASK)
                    idx_v[k][pl.ds(o, LANES)] = h.astype(jnp.int32)
                    w_v[k][pl.ds(o, LANES)] = wx[kx] * wy[ky] * wz[kz]
                return 0

            lax.fori_loop(0, CHUNK // LANES, hash_body, 0, unroll=2)

            copies = [
                pltpu.async_copy(table_hbm.at[idx_v[k]], rows_v[k], sem_g)
                for k in range(8)
            ]
            for cp in copies:
                cp.wait()

            def blend_body(g, _):
                p = g * 8 + pair
                acc = jnp.zeros((LANES,), jnp.float32)
                for k in range(8):
                    wk = plsc.load_gather(w_v[k], [p])
                    rk = plsc.load_gather(rows_v[k], [p, feat])
                    acc = acc + wk * rk
                acc_v[pl.ds(g * LANES, LANES)] = acc
                return 0

            lax.fori_loop(0, CHUNK // 8, blend_body, 0, unroll=2)

            pltpu.async_copy(
                acc_v, out_hbm.at[pl.ds(base * D, CHUNK * D)], sem_in
            ).wait()
            return 0

        lax.fori_loop(0, N_CHUNKS, chunk_body, 0)

    return sc_kernel


_SC_KERNEL = _make_sc_kernel()


def kernel(pts, voxel_features):
    xs, ys, zs = pts[:, 0], pts[:, 1], pts[:, 2]
    out_flat = _SC_KERNEL(xs, ys, zs, voxel_features)
    return out_flat.reshape(N_PTS, D)
